# Initial kernel scaffold; baseline (speedup 1.0000x reference)
import jax
import jax.numpy as jnp
from jax.experimental import pallas as pl


def kernel(pos_features, pos_matrix, neg_features, neg_matrix, batch_mask, W_self, b_self, W_hyper, b_hyper, W_score, b_score):
    raise NotImplementedError("write your pallas kernel here")


# speedup vs baseline: 2.0060x; 2.0060x over previous
import jax
import jax.numpy as jnp
from jax.experimental import pallas as pl
from jax.experimental.pallas import tpu as pltpu

N, H, F, D = 16384, 1024, 512, 512
BN = 1024
NBLK = N // BN


def _nhp_block(pf_ref, pm_ref, nf_ref, nm_ref, bm_ref,
               ws_ref, bs_ref, wh_ref, bh_ref, wsc_ref, bsc_ref,
               out_ref):
    i = pl.program_id(0)

    @pl.when(i == 0)
    def _init():
        out_ref[...] = jnp.zeros_like(out_ref)

    b = bs_ref[...] + bh_ref[...]
    ws = ws_ref[...]
    wh = wh_ref[...]
    wsc = wsc_ref[...]

    xp = jnp.dot(pf_ref[...], ws, preferred_element_type=jnp.float32)
    xp = xp + jnp.dot(pm_ref[...], wh, preferred_element_type=jnp.float32) + b
    sp = jnp.dot(jnp.maximum(xp, 0.0), wsc, preferred_element_type=jnp.float32)

    xn = jnp.dot(nf_ref[...], ws, preferred_element_type=jnp.float32)
    xn = xn + jnp.dot(nm_ref[...], wh, preferred_element_type=jnp.float32) + b
    sn = jnp.dot(jnp.maximum(xn, 0.0), wsc, preferred_element_type=jnp.float32)

    ones = jnp.ones((BN, 1), jnp.float32)
    svec = jnp.concatenate([sp, sn, ones, ones], axis=1)

    contrib = jax.lax.dot_general(
        bm_ref[...], svec, (((0,), (0,)), ((), ())),
        preferred_element_type=jnp.float32)
    out_ref[...] += contrib

    @pl.when(i == NBLK - 1)
    def _finish():
        acc = out_ref[...]
        colsum = acc[:, 2:3]
        bsc = bsc_ref[0, 0]
        pos = jax.nn.sigmoid(acc[:, 0:1] / colsum + bsc)
        neg = jax.nn.sigmoid(acc[:, 1:2] / colsum + bsc)
        out_ref[...] = jnp.concatenate([pos, neg, colsum, colsum], axis=1)


@jax.jit
def kernel(pos_features, pos_matrix, neg_features, neg_matrix, batch_mask,
           W_self, b_self, W_hyper, b_hyper, W_score, b_score):
    pf = pos_features[0]
    pm = pos_matrix[0]
    nf = neg_features[0]
    nm = neg_matrix[0]
    bm = batch_mask[0]
    bs = b_self.reshape(1, D)
    bh = b_hyper.reshape(1, D)
    bsc = b_score.reshape(1, 1)

    row_spec = pl.BlockSpec((BN, F), lambda i: (i, 0))
    mask_spec = pl.BlockSpec((BN, H), lambda i: (i, 0))
    full = lambda shape: pl.BlockSpec(shape, lambda i: (0, 0))

    out = pl.pallas_call(
        _nhp_block,
        grid=(NBLK,),
        in_specs=[row_spec, row_spec, row_spec, row_spec, mask_spec,
                  full((F, D)), full((1, D)), full((F, D)), full((1, D)),
                  full((D, 1)), full((1, 1))],
        out_specs=pl.BlockSpec((H, 4), lambda i: (0, 0)),
        out_shape=jax.ShapeDtypeStruct((H, 4), jnp.float32),
        compiler_params=pltpu.CompilerParams(
            dimension_semantics=("arbitrary",)),
    )(pf, pm, nf, nm, bm, W_self, bs, W_hyper, bh, W_score, bsc)

    return (out[:, 0:1], out[:, 1:2])


# speedup vs baseline: 2.0254x; 1.0097x over previous
import jax
import jax.numpy as jnp
from jax.experimental import pallas as pl
from jax.experimental.pallas import tpu as pltpu

N, H, F, D = 16384, 1024, 512, 512
BN = 2048
NBLK = N // BN


def _nhp_block(pf_ref, pm_ref, nf_ref, nm_ref, bm_ref,
               ws_ref, bs_ref, wh_ref, bh_ref, wsc_ref, bsc_ref,
               out_ref):
    i = pl.program_id(0)

    @pl.when(i == 0)
    def _init():
        out_ref[...] = jnp.zeros_like(out_ref)

    b = bs_ref[...] + bh_ref[...]
    ws = ws_ref[...]
    wh = wh_ref[...]
    wsc = wsc_ref[...]

    xp = jnp.dot(pf_ref[...], ws, preferred_element_type=jnp.float32)
    xp = xp + jnp.dot(pm_ref[...], wh, preferred_element_type=jnp.float32) + b
    sp = jnp.dot(jnp.maximum(xp, 0.0), wsc, preferred_element_type=jnp.float32)

    xn = jnp.dot(nf_ref[...], ws, preferred_element_type=jnp.float32)
    xn = xn + jnp.dot(nm_ref[...], wh, preferred_element_type=jnp.float32) + b
    sn = jnp.dot(jnp.maximum(xn, 0.0), wsc, preferred_element_type=jnp.float32)

    ones = jnp.ones((BN, 1), jnp.float32)
    svec = jnp.concatenate([sp, sn, ones, ones], axis=1)

    contrib = jax.lax.dot_general(
        bm_ref[...], svec, (((0,), (0,)), ((), ())),
        preferred_element_type=jnp.float32)
    out_ref[...] += contrib

    @pl.when(i == NBLK - 1)
    def _finish():
        acc = out_ref[...]
        colsum = acc[:, 2:3]
        bsc = bsc_ref[0, 0]
        pos = jax.nn.sigmoid(acc[:, 0:1] / colsum + bsc)
        neg = jax.nn.sigmoid(acc[:, 1:2] / colsum + bsc)
        out_ref[...] = jnp.concatenate([pos, neg, colsum, colsum], axis=1)


@jax.jit
def kernel(pos_features, pos_matrix, neg_features, neg_matrix, batch_mask,
           W_self, b_self, W_hyper, b_hyper, W_score, b_score):
    pf = pos_features[0]
    pm = pos_matrix[0]
    nf = neg_features[0]
    nm = neg_matrix[0]
    bm = batch_mask[0]
    bs = b_self.reshape(1, D)
    bh = b_hyper.reshape(1, D)
    bsc = b_score.reshape(1, 1)

    row_spec = pl.BlockSpec((BN, F), lambda i: (i, 0))
    mask_spec = pl.BlockSpec((BN, H), lambda i: (i, 0))
    full = lambda shape: pl.BlockSpec(shape, lambda i: (0, 0))

    out = pl.pallas_call(
        _nhp_block,
        grid=(NBLK,),
        in_specs=[row_spec, row_spec, row_spec, row_spec, mask_spec,
                  full((F, D)), full((1, D)), full((F, D)), full((1, D)),
                  full((D, 1)), full((1, 1))],
        out_specs=pl.BlockSpec((H, 4), lambda i: (0, 0)),
        out_shape=jax.ShapeDtypeStruct((H, 4), jnp.float32),
        compiler_params=pltpu.CompilerParams(
            dimension_semantics=("arbitrary",)),
    )(pf, pm, nf, nm, bm, W_self, bs, W_hyper, bh, W_score, bsc)

    return (out[:, 0:1], out[:, 1:2])
